# baseline (speedup 1.0000x reference)
import functools

import jax
import jax.numpy as jnp
from jax import lax
from jax.experimental import pallas as pl
from jax.experimental.pallas import tpu as pltpu
from jax.experimental.pallas import tpu_sc as plsc

BATCH = 16384
EMBD_DIM = 32
NUM_CORES = 2
NUM_SUBCORES = 16
NUM_WORKERS = NUM_CORES * NUM_SUBCORES
B_PER_W = BATCH // NUM_WORKERS
LANES = 16


def _lookup_body(idx_hbm, table_hbm, out_hbm, tab_v, out_v, idx_v):
    wid = lax.axis_index("s") * NUM_CORES + lax.axis_index("c")
    pltpu.sync_copy(table_hbm, tab_v)
    pltpu.sync_copy(idx_hbm.at[pl.ds(wid * B_PER_W, B_PER_W)], idx_v)

    t0a = tab_v[pl.ds(0, LANES)]
    t0b = tab_v[pl.ds(16, LANES)]
    d10a = tab_v[pl.ds(32, LANES)] - t0a
    d10b = tab_v[pl.ds(48, LANES)] - t0b
    d20a = tab_v[pl.ds(64, LANES)] - t0a
    d20b = tab_v[pl.ds(80, LANES)] - t0b

    lane_const = [jnp.full((LANES,), j, jnp.int32) for j in range(LANES)]

    @plsc.parallel_loop(0, B_PER_W // LANES, unroll=4)
    def _group(g):
        base = pl.multiple_of(g * LANES, LANES)
        vidx = idx_v[pl.ds(base, LANES)]
        vf1 = (vidx == 1).astype(jnp.float32)
        vf2 = (vidx == 2).astype(jnp.float32)
        for j in range(LANES):
            f1 = vf1.at[lane_const[j]].get(mode="promise_in_bounds")
            f2 = vf2.at[lane_const[j]].get(mode="promise_in_bounds")
            out_v[base + j, pl.ds(0, LANES)] = t0a + f1 * d10a + f2 * d20a
            out_v[base + j, pl.ds(LANES, LANES)] = t0b + f1 * d10b + f2 * d20b

    pltpu.sync_copy(out_v, out_hbm.at[pl.ds(wid * B_PER_W, B_PER_W)])


@jax.jit
def _lookup(idx_flat, table_flat):
    mesh = plsc.VectorSubcoreMesh(core_axis_name="c", subcore_axis_name="s")
    return pl.kernel(
        _lookup_body,
        out_type=jax.ShapeDtypeStruct((BATCH, EMBD_DIM), jnp.float32),
        mesh=mesh,
        compiler_params=pltpu.CompilerParams(needs_layout_passes=False),
        scratch_types=[
            pltpu.VMEM((3 * EMBD_DIM,), jnp.float32),
            pltpu.VMEM((B_PER_W, EMBD_DIM), jnp.float32),
            pltpu.VMEM((B_PER_W,), jnp.int32),
        ],
    )(idx_flat, table_flat)


def kernel(bus_type, embd_table):
    idx_flat = bus_type.astype(jnp.int32).reshape(BATCH)
    return _lookup(idx_flat, embd_table.reshape(-1))


# speedup vs baseline: 1.1148x; 1.1148x over previous
import functools

import jax
import jax.numpy as jnp
from jax import lax
from jax.experimental import pallas as pl
from jax.experimental.pallas import tpu as pltpu
from jax.experimental.pallas import tpu_sc as plsc

BATCH = 16384
EMBD_DIM = 32
NUM_CORES = 2
NUM_SUBCORES = 16
NUM_WORKERS = NUM_CORES * NUM_SUBCORES
B_PER_W = BATCH // NUM_WORKERS
LANES = 16


def _lookup_body(idx_hbm, table_hbm, out_hbm, tab_v, out_v, idx_v, sem):
    wid = lax.axis_index("s") * NUM_CORES + lax.axis_index("c")
    pltpu.sync_copy(table_hbm, tab_v)
    pltpu.sync_copy(idx_hbm.at[pl.ds(wid * B_PER_W, B_PER_W)], idx_v)

    t0a = tab_v[pl.ds(0, LANES)]
    t0b = tab_v[pl.ds(16, LANES)]
    d10a = tab_v[pl.ds(32, LANES)] - t0a
    d10b = tab_v[pl.ds(48, LANES)] - t0b
    d20a = tab_v[pl.ds(64, LANES)] - t0a
    d20b = tab_v[pl.ds(80, LANES)] - t0b

    n_q = 4
    rows_q = B_PER_W // n_q
    copies = []
    for q in range(n_q):

        @plsc.parallel_loop(0, rows_q // LANES, unroll=2)
        def _group(g, _q=q):
            base = pl.multiple_of(_q * rows_q + g * LANES, LANES)
            vidx = idx_v[pl.ds(base, LANES)]
            vf1 = (vidx == 1).astype(jnp.float32)
            vf2 = (vidx == 2).astype(jnp.float32)
            for j in range(LANES):
                f1 = vf1[j]
                f2 = vf2[j]
                out_v[base + j, pl.ds(0, LANES)] = t0a + f1 * d10a + f2 * d20a
                out_v[base + j, pl.ds(LANES, LANES)] = t0b + f1 * d10b + f2 * d20b

        copies.append(
            pltpu.async_copy(
                out_v.at[pl.ds(q * rows_q, rows_q)],
                out_hbm.at[pl.ds(wid * B_PER_W + q * rows_q, rows_q)],
                sem,
            )
        )
    for c in copies:
        c.wait()


@jax.jit
def _lookup(idx_flat, table_flat):
    mesh = plsc.VectorSubcoreMesh(core_axis_name="c", subcore_axis_name="s")
    return pl.kernel(
        _lookup_body,
        out_type=jax.ShapeDtypeStruct((BATCH, EMBD_DIM), jnp.float32),
        mesh=mesh,
        compiler_params=pltpu.CompilerParams(needs_layout_passes=False),
        scratch_types=[
            pltpu.VMEM((3 * EMBD_DIM,), jnp.float32),
            pltpu.VMEM((B_PER_W, EMBD_DIM), jnp.float32),
            pltpu.VMEM((B_PER_W,), jnp.int32),
            pltpu.SemaphoreType.DMA,
        ],
    )(idx_flat, table_flat)


def kernel(bus_type, embd_table):
    idx_flat = bus_type.astype(jnp.int32).reshape(BATCH)
    return _lookup(idx_flat, embd_table.reshape(-1))


# speedup vs baseline: 1.2073x; 1.0830x over previous
import functools

import jax
import jax.numpy as jnp
from jax import lax
from jax.experimental import pallas as pl
from jax.experimental.pallas import tpu as pltpu
from jax.experimental.pallas import tpu_sc as plsc

BATCH = 16384
EMBD_DIM = 32
NUM_CORES = 2
NUM_SUBCORES = 16
NUM_WORKERS = NUM_CORES * NUM_SUBCORES
B_PER_W = BATCH // NUM_WORKERS
LANES = 16


def _lookup_body(idx_hbm, table_hbm, out_hbm, tab_v, out_v, idx_v, sem):
    wid = lax.axis_index("s") * NUM_CORES + lax.axis_index("c")
    pltpu.sync_copy(table_hbm, tab_v)
    pltpu.sync_copy(idx_hbm.at[pl.ds(wid * B_PER_W, B_PER_W)], idx_v)

    t0a = tab_v[pl.ds(0, LANES)]
    t0b = tab_v[pl.ds(16, LANES)]
    d10a = tab_v[pl.ds(32, LANES)] - t0a
    d10b = tab_v[pl.ds(48, LANES)] - t0b
    d20a = tab_v[pl.ds(64, LANES)] - t0a
    d20b = tab_v[pl.ds(80, LANES)] - t0b

    @plsc.parallel_loop(0, B_PER_W // LANES, unroll=1)
    def _group(g):
        base = pl.multiple_of(g * LANES, LANES)
        vidx = idx_v[pl.ds(base, LANES)]
        vf1 = (vidx == 1).astype(jnp.float32)
        vf2 = (vidx == 2).astype(jnp.float32)
        for j in range(LANES):
            f1 = vf1[j]
            f2 = vf2[j]
            out_v[base + j, pl.ds(0, LANES)] = t0a + f1 * d10a + f2 * d20a
            out_v[base + j, pl.ds(LANES, LANES)] = t0b + f1 * d10b + f2 * d20b

    pltpu.sync_copy(out_v, out_hbm.at[pl.ds(wid * B_PER_W, B_PER_W)])


@jax.jit
def _lookup(idx_flat, table_flat):
    mesh = plsc.VectorSubcoreMesh(core_axis_name="c", subcore_axis_name="s")
    return pl.kernel(
        _lookup_body,
        out_type=jax.ShapeDtypeStruct((BATCH, EMBD_DIM), jnp.float32),
        mesh=mesh,
        compiler_params=pltpu.CompilerParams(needs_layout_passes=False),
        scratch_types=[
            pltpu.VMEM((3 * EMBD_DIM,), jnp.float32),
            pltpu.VMEM((B_PER_W, EMBD_DIM), jnp.float32),
            pltpu.VMEM((B_PER_W,), jnp.int32),
            pltpu.SemaphoreType.DMA,
        ],
    )(idx_flat, table_flat)


def kernel(bus_type, embd_table):
    idx_flat = bus_type.astype(jnp.int32).reshape(BATCH)
    return _lookup(idx_flat, embd_table.reshape(-1))
